# Initial kernel scaffold; baseline (speedup 1.0000x reference)
import jax
import jax.numpy as jnp
from jax.experimental import pallas as pl


def kernel(features, edge_index, W_enc, W_layers, W_fc, b_fc):
    raise NotImplementedError("write your pallas kernel here")


# speedup vs baseline: 4.4498x; 4.4498x over previous
import functools

import jax
import jax.numpy as jnp
from jax import lax
from jax.experimental import pallas as pl
from jax.experimental.pallas import tpu as pltpu
from jax.experimental.pallas import tpu_sc as plsc

N = 10000
E = 320000
D = 128
DH = D // 2
T = 4
VTH = 1.0
STEP = 0.1

NC = 2
NS = 16
EPT = E // NS
C = 128
NFULL = EPT // C
CTAIL = EPT - NFULL * C
ROWS_PT = (N // NS) // 8 * 8
ROWS_REM = N - NS * ROWS_PT
REM0 = NS * ROWS_PT

_MESH = plsc.VectorSubcoreMesh(
    core_axis_name="c", subcore_axis_name="s", num_cores=NC, num_subcores=NS
)


def _make_agg(with_deg: bool):
    out_type = [
        jax.ShapeDtypeStruct((N, DH), jnp.float32),
        jax.ShapeDtypeStruct((N, DH), jnp.float32),
    ]
    scratch = [
        pltpu.VMEM((C,), jnp.int32),
        pltpu.VMEM((C,), jnp.int32),
        pltpu.VMEM((C, DH), jnp.float32),
        pltpu.VMEM((CTAIL,), jnp.int32),
        pltpu.VMEM((CTAIL,), jnp.int32),
        pltpu.VMEM((CTAIL, DH), jnp.float32),
        pltpu.VMEM((ROWS_PT, DH), jnp.float32),
        pltpu.VMEM_SHARED((N, DH), jnp.float32),
        pltpu.SemaphoreType.DMA,
    ]
    if with_deg:
        out_type.append(jax.ShapeDtypeStruct((N, 16), jnp.float32))
        scratch += [
            pltpu.VMEM((C, 16), jnp.float32),
            pltpu.VMEM((ROWS_PT, 16), jnp.float32),
            pltpu.VMEM_SHARED((N, 16), jnp.float32),
        ]

    @functools.partial(
        pl.kernel, out_type=out_type, mesh=_MESH, scratch_types=scratch,
        compiler_params=pltpu.CompilerParams(use_tc_tiling_on_sc=False),
    )
    def agg(*refs):
        if with_deg:
            (xlo_hbm, xhi_hbm, src_hbm, dst_hbm, zeros_hbm, zeros16_hbm,
             ones_hbm, outlo_hbm, outhi_hbm, deg_hbm,
             srci, dsti, rows, srci_t, dsti_t, rows_t, wb, acc, sem,
             ones_v, degwb, dacc) = refs
        else:
            (xlo_hbm, xhi_hbm, src_hbm, dst_hbm, zeros_hbm,
             outlo_hbm, outhi_hbm,
             srci, dsti, rows, srci_t, dsti_t, rows_t, wb, acc, sem) = refs

        c = lax.axis_index("c")
        s = lax.axis_index("s")
        r0 = s * ROWS_PT
        edge_base = s * EPT
        deg_here = with_deg

        pltpu.sync_copy(zeros_hbm.at[pl.ds(r0, ROWS_PT)],
                        acc.at[pl.ds(r0, ROWS_PT)])
        if deg_here:
            pltpu.sync_copy(ones_hbm, ones_v)

            @pl.when(c == 0)
            def _zero_deg():
                pltpu.sync_copy(zeros16_hbm.at[pl.ds(r0, ROWS_PT)],
                                dacc.at[pl.ds(r0, ROWS_PT)])

        @pl.when(s == NS - 1)
        def _zero_rem():
            pltpu.sync_copy(zeros_hbm.at[pl.ds(REM0, ROWS_REM)],
                            acc.at[pl.ds(REM0, ROWS_REM)])
            if with_deg:
                @pl.when(c == 0)
                def _zero_deg_rem():
                    pltpu.sync_copy(zeros16_hbm.at[pl.ds(REM0, ROWS_REM)],
                                    dacc.at[pl.ds(REM0, ROWS_REM)])

        plsc.subcore_barrier()

        def run_core(x_hbm):
            def chunk(k, _):
                base = edge_base + k * C
                pltpu.sync_copy(src_hbm.at[pl.ds(base, C)], srci)
                pltpu.sync_copy(dst_hbm.at[pl.ds(base, C)], dsti)
                pltpu.async_copy(x_hbm.at[srci], rows, sem).wait()
                pltpu.sync_copy(rows, acc.at[dsti], add=True)
                if deg_here:
                    @pl.when(c == 0)
                    def _deg_add():
                        pltpu.sync_copy(ones_v, dacc.at[dsti], add=True)
                return 0

            lax.fori_loop(0, NFULL, chunk, 0)

            if CTAIL:
                base = edge_base + NFULL * C
                pltpu.sync_copy(src_hbm.at[pl.ds(base, CTAIL)], srci_t)
                pltpu.sync_copy(dst_hbm.at[pl.ds(base, CTAIL)], dsti_t)
                pltpu.async_copy(x_hbm.at[srci_t], rows_t, sem).wait()
                pltpu.sync_copy(rows_t, acc.at[dsti_t], add=True)
                if deg_here:
                    @pl.when(c == 0)
                    def _deg_add_t():
                        pltpu.sync_copy(ones_v.at[pl.ds(0, CTAIL)],
                                        dacc.at[dsti_t], add=True)

        @pl.when(c == 0)
        def _core0():
            run_core(xlo_hbm)

        @pl.when(c == 1)
        def _core1():
            run_core(xhi_hbm)

        plsc.subcore_barrier()

        def write_out(out_hbm):
            pltpu.sync_copy(acc.at[pl.ds(r0, ROWS_PT)], wb)
            pltpu.sync_copy(wb, out_hbm.at[pl.ds(r0, ROWS_PT)])

            @pl.when(s == NS - 1)
            def _wb_rem():
                pltpu.sync_copy(acc.at[pl.ds(REM0, ROWS_REM)],
                                rows_t.at[pl.ds(0, ROWS_REM)])
                pltpu.sync_copy(rows_t.at[pl.ds(0, ROWS_REM)],
                                out_hbm.at[pl.ds(REM0, ROWS_REM)])

        @pl.when(c == 0)
        def _wb0():
            write_out(outlo_hbm)
            if deg_here:
                pltpu.sync_copy(dacc.at[pl.ds(r0, ROWS_PT)], degwb)
                pltpu.sync_copy(degwb, deg_hbm.at[pl.ds(r0, ROWS_PT)])

                @pl.when(s == NS - 1)
                def _wb_deg_rem():
                    pltpu.sync_copy(dacc.at[pl.ds(REM0, ROWS_REM)],
                                    ones_v.at[pl.ds(0, ROWS_REM)])
                    pltpu.sync_copy(ones_v.at[pl.ds(0, ROWS_REM)],
                                    deg_hbm.at[pl.ds(REM0, ROWS_REM)])

        @pl.when(c == 1)
        def _wb1():
            write_out(outhi_hbm)

    return agg


_agg_deg = _make_agg(True)
_agg = _make_agg(False)


B = 1000


def _if_rate(lo_ref, hi_ref, deg_ref):
    t = jnp.concatenate((lo_ref[...], hi_ref[...]), axis=1)
    t = t / jnp.maximum(deg_ref[:, 0:1], 1.0)
    v = jnp.zeros_like(t)
    ssum = jnp.zeros_like(t)
    for _ in range(T):
        v = v + t
        sp = (v >= VTH).astype(t.dtype)
        v = v - sp * VTH
        ssum = ssum + sp
    return ssum * (1.0 / T)


def _enc_body(f_ref, w_ref, hlo_ref, hhi_ref):
    h = jnp.dot(f_ref[...], w_ref[...], preferred_element_type=jnp.float32)
    hlo_ref[...] = h[:, :DH]
    hhi_ref[...] = h[:, DH:]


def _mid_body(lo_ref, hi_ref, deg_ref, w_ref, x_ref, hlo_ref, hhi_ref):
    x = _if_rate(lo_ref, hi_ref, deg_ref)
    x_ref[...] = x
    h = jnp.dot(x, w_ref[...], preferred_element_type=jnp.float32)
    hlo_ref[...] = h[:, :DH]
    hhi_ref[...] = h[:, DH:]


def _fin_body(lo_ref, hi_ref, deg_ref, x0_ref, x1_ref, wfc_ref, bfc_ref,
              out_ref):
    x2 = _if_rate(lo_ref, hi_ref, deg_ref)
    z = (x0_ref[...] + x1_ref[...] + x2) * STEP
    out_ref[...] = jnp.dot(z, wfc_ref[...],
                           preferred_element_type=jnp.float32) + bfc_ref[...]


_half_spec = pl.BlockSpec((B, DH), lambda i: (i, 0))
_full_spec = pl.BlockSpec((B, D), lambda i: (i, 0))
_deg_spec = pl.BlockSpec((B, 16), lambda i: (i, 0))
_w_spec = pl.BlockSpec((D, D), lambda i: (0, 0))
_half_out = [
    jax.ShapeDtypeStruct((N, DH), jnp.float32),
    jax.ShapeDtypeStruct((N, DH), jnp.float32),
]

_enc_tc = pl.pallas_call(
    _enc_body,
    grid=(N // B,),
    in_specs=[_full_spec, _w_spec],
    out_specs=[_half_spec, _half_spec],
    out_shape=_half_out,
)

_mid_tc = pl.pallas_call(
    _mid_body,
    grid=(N // B,),
    in_specs=[_half_spec, _half_spec, _deg_spec, _w_spec],
    out_specs=[_full_spec, _half_spec, _half_spec],
    out_shape=[jax.ShapeDtypeStruct((N, D), jnp.float32)] + _half_out,
)

_fin_tc = pl.pallas_call(
    _fin_body,
    grid=(N // B,),
    in_specs=[
        _half_spec,
        _half_spec,
        _deg_spec,
        _full_spec,
        _full_spec,
        pl.BlockSpec((D, 40), lambda i: (0, 0)),
        pl.BlockSpec((1, 40), lambda i: (0, 0)),
    ],
    out_specs=pl.BlockSpec((B, 40), lambda i: (i, 0)),
    out_shape=jax.ShapeDtypeStruct((N, 40), jnp.float32),
)


def kernel(features, edge_index, W_enc, W_layers, W_fc, b_fc):
    src = edge_index[0]
    dst = edge_index[1]
    zeros = jnp.zeros((N, DH), jnp.float32)
    zeros16 = jnp.zeros((N, 16), jnp.float32)
    ones = jnp.ones((C, 16), jnp.float32)

    h0_lo, h0_hi = _enc_tc(features, W_enc)
    s0_lo, s0_hi, deg = _agg_deg(h0_lo, h0_hi, src, dst, zeros, zeros16, ones)
    x0, h1_lo, h1_hi = _mid_tc(s0_lo, s0_hi, deg, W_layers[0])
    s1_lo, s1_hi = _agg(h1_lo, h1_hi, src, dst, zeros)
    x1, h2_lo, h2_hi = _mid_tc(s1_lo, s1_hi, deg, W_layers[1])
    s2_lo, s2_hi = _agg(h2_lo, h2_hi, src, dst, zeros)
    logits = _fin_tc(s2_lo, s2_hi, deg, x0, x1, W_fc, b_fc.reshape(1, 40))
    return logits


# speedup vs baseline: 7.3643x; 1.6550x over previous
import functools

import jax
import jax.numpy as jnp
from jax import lax
from jax.experimental import pallas as pl
from jax.experimental.pallas import tpu as pltpu
from jax.experimental.pallas import tpu_sc as plsc

N = 10000
E = 320000
D = 128
DH = D // 2
T = 4
VTH = 1.0
STEP = 0.1

NC = 2
NS = 16
C = 128
NCH = -(-E // (NS * C))
E_PAD = NS * NCH * C
EROWS = E_PAD // C
NA = N + 16
ROWS_PT = (N // NS) // 8 * 8
ROWS_REM = N - NS * ROWS_PT
REM0 = NS * ROWS_PT

_MESH = plsc.VectorSubcoreMesh(
    core_axis_name="c", subcore_axis_name="s", num_cores=NC, num_subcores=NS
)


def _make_agg(with_deg: bool):
    out_type = [
        jax.ShapeDtypeStruct((N, DH), jnp.float32),
        jax.ShapeDtypeStruct((N, DH), jnp.float32),
    ]
    scratch = [
        pltpu.VMEM((NCH, C), jnp.int32),
        pltpu.VMEM((NCH, C), jnp.int32),
        pltpu.VMEM((C, DH), jnp.float32),
        pltpu.VMEM((C, DH), jnp.float32),
        pltpu.VMEM_SHARED((NA, DH), jnp.float32),
        pltpu.SemaphoreType.DMA,
        pltpu.SemaphoreType.DMA,
    ]
    if with_deg:
        out_type.append(jax.ShapeDtypeStruct((N, 16), jnp.float32))
        scratch += [
            pltpu.VMEM((C, 16), jnp.float32),
            pltpu.VMEM_SHARED((NA, 16), jnp.float32),
        ]

    @functools.partial(
        pl.kernel, out_type=out_type, mesh=_MESH, scratch_types=scratch,
        compiler_params=pltpu.CompilerParams(use_tc_tiling_on_sc=False),
    )
    def agg(*refs):
        if with_deg:
            (xlo_hbm, xhi_hbm, src_hbm, dst_hbm, zeros_hbm, zeros16_hbm,
             ones_hbm, outlo_hbm, outhi_hbm, deg_hbm,
             srci, dsti, rowsa, rowsb, acc, sema, semb,
             ones_v, dacc) = refs
        else:
            (xlo_hbm, xhi_hbm, src_hbm, dst_hbm, zeros_hbm,
             outlo_hbm, outhi_hbm,
             srci, dsti, rowsa, rowsb, acc, sema, semb) = refs

        c = lax.axis_index("c")
        s = lax.axis_index("s")
        r0 = s * ROWS_PT
        row_base = s * NCH
        deg_here = with_deg

        pltpu.sync_copy(src_hbm.at[pl.ds(row_base, NCH)], srci)
        pltpu.sync_copy(dst_hbm.at[pl.ds(row_base, NCH)], dsti)

        pltpu.sync_copy(zeros_hbm.at[pl.ds(r0, ROWS_PT)],
                        acc.at[pl.ds(r0, ROWS_PT)])
        if deg_here:
            pltpu.sync_copy(ones_hbm, ones_v)

            @pl.when(c == 0)
            def _zero_deg():
                pltpu.sync_copy(zeros16_hbm.at[pl.ds(r0, ROWS_PT)],
                                dacc.at[pl.ds(r0, ROWS_PT)])

        @pl.when(s == NS - 1)
        def _zero_rem():
            pltpu.sync_copy(zeros_hbm.at[pl.ds(REM0, ROWS_REM)],
                            acc.at[pl.ds(REM0, ROWS_REM)])
            if with_deg:
                @pl.when(c == 0)
                def _zero_deg_rem():
                    pltpu.sync_copy(zeros16_hbm.at[pl.ds(REM0, ROWS_REM)],
                                    dacc.at[pl.ds(REM0, ROWS_REM)])

        plsc.subcore_barrier()

        def run_core(x_hbm):
            def gather(k, buf, sem):
                pltpu.async_copy(x_hbm.at[srci.at[k]], buf, sem)

            def gwait(k, buf, sem):
                pltpu.make_async_copy(x_hbm.at[srci.at[k]], buf, sem).wait()

            def scat(k, buf):
                pltpu.sync_copy(buf, acc.at[dsti.at[k]], add=True)
                if deg_here:
                    @pl.when(c == 0)
                    def _deg_add():
                        pltpu.sync_copy(ones_v, dacc.at[dsti.at[k]], add=True)

            gather(0, rowsa, sema)

            def pair(j2, _):
                k0 = j2 * 2
                gwait(k0, rowsa, sema)
                gather(k0 + 1, rowsb, semb)
                scat(k0, rowsa)
                gwait(k0 + 1, rowsb, semb)
                gather(k0 + 2, rowsa, sema)
                scat(k0 + 1, rowsb)
                return 0

            lax.fori_loop(0, (NCH - 1) // 2, pair, 0)
            gwait(NCH - 1, rowsa, sema)
            scat(NCH - 1, rowsa)

        @pl.when(c == 0)
        def _core0():
            run_core(xlo_hbm)

        @pl.when(c == 1)
        def _core1():
            run_core(xhi_hbm)

        plsc.subcore_barrier()

        def write_out(out_hbm):
            pltpu.sync_copy(acc.at[pl.ds(r0, ROWS_PT)],
                            out_hbm.at[pl.ds(r0, ROWS_PT)])

            @pl.when(s == NS - 1)
            def _wb_rem():
                pltpu.sync_copy(acc.at[pl.ds(REM0, ROWS_REM)],
                                out_hbm.at[pl.ds(REM0, ROWS_REM)])

        @pl.when(c == 0)
        def _wb0():
            write_out(outlo_hbm)
            if deg_here:
                pltpu.sync_copy(dacc.at[pl.ds(r0, ROWS_PT)],
                                deg_hbm.at[pl.ds(r0, ROWS_PT)])

                @pl.when(s == NS - 1)
                def _wb_deg_rem():
                    pltpu.sync_copy(dacc.at[pl.ds(REM0, ROWS_REM)],
                                    deg_hbm.at[pl.ds(REM0, ROWS_REM)])

        @pl.when(c == 1)
        def _wb1():
            write_out(outhi_hbm)

    return agg


_agg_deg = _make_agg(True)
_agg = _make_agg(False)


B = 1000


def _if_rate(lo_ref, hi_ref, deg_ref):
    t = jnp.concatenate((lo_ref[...], hi_ref[...]), axis=1)
    t = t / jnp.maximum(deg_ref[:, 0:1], 1.0)
    v = jnp.zeros_like(t)
    ssum = jnp.zeros_like(t)
    for _ in range(T):
        v = v + t
        sp = (v >= VTH).astype(t.dtype)
        v = v - sp * VTH
        ssum = ssum + sp
    return ssum * (1.0 / T)


def _enc_body(f_ref, w_ref, hlo_ref, hhi_ref):
    h = jnp.dot(f_ref[...], w_ref[...], preferred_element_type=jnp.float32)
    hlo_ref[...] = h[:, :DH]
    hhi_ref[...] = h[:, DH:]


def _mid_body(lo_ref, hi_ref, deg_ref, w_ref, x_ref, hlo_ref, hhi_ref):
    x = _if_rate(lo_ref, hi_ref, deg_ref)
    x_ref[...] = x
    h = jnp.dot(x, w_ref[...], preferred_element_type=jnp.float32)
    hlo_ref[...] = h[:, :DH]
    hhi_ref[...] = h[:, DH:]


def _fin_body(lo_ref, hi_ref, deg_ref, x0_ref, x1_ref, wfc_ref, bfc_ref,
              out_ref):
    x2 = _if_rate(lo_ref, hi_ref, deg_ref)
    z = (x0_ref[...] + x1_ref[...] + x2) * STEP
    out_ref[...] = jnp.dot(z, wfc_ref[...],
                           preferred_element_type=jnp.float32) + bfc_ref[...]


_half_spec = pl.BlockSpec((B, DH), lambda i: (i, 0))
_full_spec = pl.BlockSpec((B, D), lambda i: (i, 0))
_deg_spec = pl.BlockSpec((B, 16), lambda i: (i, 0))
_w_spec = pl.BlockSpec((D, D), lambda i: (0, 0))
_half_out = [
    jax.ShapeDtypeStruct((N, DH), jnp.float32),
    jax.ShapeDtypeStruct((N, DH), jnp.float32),
]

_enc_tc = pl.pallas_call(
    _enc_body,
    grid=(N // B,),
    in_specs=[_full_spec, _w_spec],
    out_specs=[_half_spec, _half_spec],
    out_shape=_half_out,
)

_mid_tc = pl.pallas_call(
    _mid_body,
    grid=(N // B,),
    in_specs=[_half_spec, _half_spec, _deg_spec, _w_spec],
    out_specs=[_full_spec, _half_spec, _half_spec],
    out_shape=[jax.ShapeDtypeStruct((N, D), jnp.float32)] + _half_out,
)

_fin_tc = pl.pallas_call(
    _fin_body,
    grid=(N // B,),
    in_specs=[
        _half_spec,
        _half_spec,
        _deg_spec,
        _full_spec,
        _full_spec,
        pl.BlockSpec((D, 40), lambda i: (0, 0)),
        pl.BlockSpec((1, 40), lambda i: (0, 0)),
    ],
    out_specs=pl.BlockSpec((B, 40), lambda i: (i, 0)),
    out_shape=jax.ShapeDtypeStruct((N, 40), jnp.float32),
)


def kernel(features, edge_index, W_enc, W_layers, W_fc, b_fc):
    src2d = jnp.concatenate(
        [edge_index[0], jnp.zeros((E_PAD - E,), jnp.int32)]).reshape(EROWS, C)
    dst2d = jnp.concatenate(
        [edge_index[1], jnp.full((E_PAD - E,), N, jnp.int32)]).reshape(EROWS, C)
    zeros = jnp.zeros((N, DH), jnp.float32)
    zeros16 = jnp.zeros((N, 16), jnp.float32)
    ones = jnp.ones((C, 16), jnp.float32)

    h0_lo, h0_hi = _enc_tc(features, W_enc)
    s0_lo, s0_hi, deg = _agg_deg(h0_lo, h0_hi, src2d, dst2d, zeros, zeros16,
                                 ones)
    x0, h1_lo, h1_hi = _mid_tc(s0_lo, s0_hi, deg, W_layers[0])
    s1_lo, s1_hi = _agg(h1_lo, h1_hi, src2d, dst2d, zeros)
    x1, h2_lo, h2_hi = _mid_tc(s1_lo, s1_hi, deg, W_layers[1])
    s2_lo, s2_hi = _agg(h2_lo, h2_hi, src2d, dst2d, zeros)
    logits = _fin_tc(s2_lo, s2_hi, deg, x0, x1, W_fc, b_fc.reshape(1, 40))
    return logits


# speedup vs baseline: 9.7311x; 1.3214x over previous
import functools

import jax
import jax.numpy as jnp
from jax import lax
from jax.experimental import pallas as pl
from jax.experimental.pallas import tpu as pltpu
from jax.experimental.pallas import tpu_sc as plsc

N = 10000
E = 320000
D = 128
DH = D // 2
T = 4
VTH = 1.0
STEP = 0.1

NC = 2
NS = 16
C = 128
NCH = -(-E // (NS * C))
E_PAD = NS * NCH * C
EROWS = E_PAD // C
NA = N + 16
ROWS_PT = (N // NS) // 8 * 8
ROWS_REM = N - NS * ROWS_PT
REM0 = NS * ROWS_PT

_MESH = plsc.VectorSubcoreMesh(
    core_axis_name="c", subcore_axis_name="s", num_cores=NC, num_subcores=NS
)


def _make_agg(with_deg: bool):
    out_type = [
        jax.ShapeDtypeStruct((N, DH), jnp.float32),
        jax.ShapeDtypeStruct((N, DH), jnp.float32),
    ]
    scratch = [
        pltpu.VMEM((NCH, C), jnp.int32),
        pltpu.VMEM((NCH, C), jnp.int32),
        pltpu.VMEM((C, DH), jnp.float32),
        pltpu.VMEM((C, DH), jnp.float32),
        pltpu.VMEM((C, DH), jnp.float32),
        pltpu.VMEM((C, DH), jnp.float32),
        pltpu.VMEM_SHARED((NA, DH), jnp.float32),
        pltpu.SemaphoreType.DMA,
        pltpu.SemaphoreType.DMA,
        pltpu.SemaphoreType.DMA,
        pltpu.SemaphoreType.DMA,
        pltpu.SemaphoreType.DMA,
        pltpu.SemaphoreType.DMA,
        pltpu.SemaphoreType.DMA,
        pltpu.SemaphoreType.DMA,
    ]
    if with_deg:
        out_type.append(jax.ShapeDtypeStruct((N, 16), jnp.float32))
        scratch += [
            pltpu.VMEM((C, 16), jnp.float32),
            pltpu.VMEM_SHARED((NA, 16), jnp.float32),
        ]

    @functools.partial(
        pl.kernel, out_type=out_type, mesh=_MESH, scratch_types=scratch,
        compiler_params=pltpu.CompilerParams(use_tc_tiling_on_sc=False),
    )
    def agg(*refs):
        if with_deg:
            (xlo_hbm, xhi_hbm, src_hbm, dst_hbm, zeros_hbm, zeros16_hbm,
             ones_hbm, outlo_hbm, outhi_hbm, deg_hbm,
             srci, dsti, rows0, rows1, rows2, rows3, acc,
             gsem0, gsem1, gsem2, gsem3, ssem0, ssem1, ssem2, ssem3,
             ones_v, dacc) = refs
        else:
            (xlo_hbm, xhi_hbm, src_hbm, dst_hbm, zeros_hbm,
             outlo_hbm, outhi_hbm,
             srci, dsti, rows0, rows1, rows2, rows3, acc,
             gsem0, gsem1, gsem2, gsem3, ssem0, ssem1, ssem2, ssem3) = refs

        c = lax.axis_index("c")
        s = lax.axis_index("s")
        r0 = s * ROWS_PT
        row_base = s * NCH
        deg_here = with_deg

        pltpu.sync_copy(src_hbm.at[pl.ds(row_base, NCH)], srci)
        pltpu.sync_copy(dst_hbm.at[pl.ds(row_base, NCH)], dsti)

        pltpu.sync_copy(zeros_hbm.at[pl.ds(r0, ROWS_PT)],
                        acc.at[pl.ds(r0, ROWS_PT)])
        if deg_here:
            pltpu.sync_copy(ones_hbm, ones_v)

            @pl.when(c == 0)
            def _zero_deg():
                pltpu.sync_copy(zeros16_hbm.at[pl.ds(r0, ROWS_PT)],
                                dacc.at[pl.ds(r0, ROWS_PT)])

        @pl.when(s == NS - 1)
        def _zero_rem():
            pltpu.sync_copy(zeros_hbm.at[pl.ds(REM0, ROWS_REM)],
                            acc.at[pl.ds(REM0, ROWS_REM)])
            if with_deg:
                @pl.when(c == 0)
                def _zero_deg_rem():
                    pltpu.sync_copy(zeros16_hbm.at[pl.ds(REM0, ROWS_REM)],
                                    dacc.at[pl.ds(REM0, ROWS_REM)])

        plsc.subcore_barrier()

        def run_core(x_hbm):
            rows = [rows0, rows1, rows2, rows3]
            gsem = [gsem0, gsem1, gsem2, gsem3]
            ssem = [ssem0, ssem1, ssem2, ssem3]

            def gather(k, b):
                pltpu.async_copy(x_hbm.at[srci.at[k]], rows[b], gsem[b])

            def gwait(k, b):
                pltpu.make_async_copy(x_hbm.at[srci.at[k]], rows[b],
                                      gsem[b]).wait()

            def scat(k, b):
                pltpu.async_copy(rows[b], acc.at[dsti.at[k]], ssem[b],
                                 add=True)
                if deg_here:
                    @pl.when(c == 0)
                    def _deg_add():
                        pltpu.sync_copy(ones_v, dacc.at[dsti.at[k]], add=True)

            def swait(k, b):
                pltpu.make_async_copy(rows[b], acc.at[dsti.at[k]],
                                      ssem[b]).wait()

            gather(0, 0)
            gather(1, 1)
            gather(2, 2)
            gwait(0, 0)
            scat(0, 0)
            gather(3, 3)
            gwait(1, 1)
            scat(1, 1)

            def quad(j, _):
                for b in range(4):
                    i = j * 4 + b
                    swait(i - 4, b)
                    gather(i, b)
                    b2 = (b + 2) % 4
                    gwait(i - 2, b2)
                    scat(i - 2, b2)
                return 0

            lax.fori_loop(1, NCH // 4, quad, 0)
            swait(NCH - 5, 0)
            gather(NCH - 1, 0)
            gwait(NCH - 3, 2)
            scat(NCH - 3, 2)
            gwait(NCH - 2, 3)
            scat(NCH - 2, 3)
            gwait(NCH - 1, 0)
            scat(NCH - 1, 0)
            swait(NCH - 4, 1)
            swait(NCH - 3, 2)
            swait(NCH - 2, 3)
            swait(NCH - 1, 0)

        @pl.when(c == 0)
        def _core0():
            run_core(xlo_hbm)

        @pl.when(c == 1)
        def _core1():
            run_core(xhi_hbm)

        plsc.subcore_barrier()

        def write_out(out_hbm):
            pltpu.sync_copy(acc.at[pl.ds(r0, ROWS_PT)],
                            out_hbm.at[pl.ds(r0, ROWS_PT)])

            @pl.when(s == NS - 1)
            def _wb_rem():
                pltpu.sync_copy(acc.at[pl.ds(REM0, ROWS_REM)],
                                out_hbm.at[pl.ds(REM0, ROWS_REM)])

        @pl.when(c == 0)
        def _wb0():
            write_out(outlo_hbm)
            if deg_here:
                pltpu.sync_copy(dacc.at[pl.ds(r0, ROWS_PT)],
                                deg_hbm.at[pl.ds(r0, ROWS_PT)])

                @pl.when(s == NS - 1)
                def _wb_deg_rem():
                    pltpu.sync_copy(dacc.at[pl.ds(REM0, ROWS_REM)],
                                    deg_hbm.at[pl.ds(REM0, ROWS_REM)])

        @pl.when(c == 1)
        def _wb1():
            write_out(outhi_hbm)

    return agg


_agg_deg = _make_agg(True)
_agg = _make_agg(False)


B = 1000


def _if_rate(lo_ref, hi_ref, deg_ref):
    t = jnp.concatenate((lo_ref[...], hi_ref[...]), axis=1)
    t = t / jnp.maximum(deg_ref[:, 0:1], 1.0)
    v = jnp.zeros_like(t)
    ssum = jnp.zeros_like(t)
    for _ in range(T):
        v = v + t
        sp = (v >= VTH).astype(t.dtype)
        v = v - sp * VTH
        ssum = ssum + sp
    return ssum * (1.0 / T)


def _enc_body(f_ref, w_ref, hlo_ref, hhi_ref):
    h = jnp.dot(f_ref[...], w_ref[...], preferred_element_type=jnp.float32)
    hlo_ref[...] = h[:, :DH]
    hhi_ref[...] = h[:, DH:]


def _mid_body(lo_ref, hi_ref, deg_ref, w_ref, x_ref, hlo_ref, hhi_ref):
    x = _if_rate(lo_ref, hi_ref, deg_ref)
    x_ref[...] = x
    h = jnp.dot(x, w_ref[...], preferred_element_type=jnp.float32)
    hlo_ref[...] = h[:, :DH]
    hhi_ref[...] = h[:, DH:]


def _fin_body(lo_ref, hi_ref, deg_ref, x0_ref, x1_ref, wfc_ref, bfc_ref,
              out_ref):
    x2 = _if_rate(lo_ref, hi_ref, deg_ref)
    z = (x0_ref[...] + x1_ref[...] + x2) * STEP
    out_ref[...] = jnp.dot(z, wfc_ref[...],
                           preferred_element_type=jnp.float32) + bfc_ref[...]


_half_spec = pl.BlockSpec((B, DH), lambda i: (i, 0))
_full_spec = pl.BlockSpec((B, D), lambda i: (i, 0))
_deg_spec = pl.BlockSpec((B, 16), lambda i: (i, 0))
_w_spec = pl.BlockSpec((D, D), lambda i: (0, 0))
_half_out = [
    jax.ShapeDtypeStruct((N, DH), jnp.float32),
    jax.ShapeDtypeStruct((N, DH), jnp.float32),
]

_enc_tc = pl.pallas_call(
    _enc_body,
    grid=(N // B,),
    in_specs=[_full_spec, _w_spec],
    out_specs=[_half_spec, _half_spec],
    out_shape=_half_out,
)

_mid_tc = pl.pallas_call(
    _mid_body,
    grid=(N // B,),
    in_specs=[_half_spec, _half_spec, _deg_spec, _w_spec],
    out_specs=[_full_spec, _half_spec, _half_spec],
    out_shape=[jax.ShapeDtypeStruct((N, D), jnp.float32)] + _half_out,
)

_fin_tc = pl.pallas_call(
    _fin_body,
    grid=(N // B,),
    in_specs=[
        _half_spec,
        _half_spec,
        _deg_spec,
        _full_spec,
        _full_spec,
        pl.BlockSpec((D, 40), lambda i: (0, 0)),
        pl.BlockSpec((1, 40), lambda i: (0, 0)),
    ],
    out_specs=pl.BlockSpec((B, 40), lambda i: (i, 0)),
    out_shape=jax.ShapeDtypeStruct((N, 40), jnp.float32),
)


def kernel(features, edge_index, W_enc, W_layers, W_fc, b_fc):
    src2d = jnp.concatenate(
        [edge_index[0], jnp.zeros((E_PAD - E,), jnp.int32)]).reshape(EROWS, C)
    dst2d = jnp.concatenate(
        [edge_index[1], jnp.full((E_PAD - E,), N, jnp.int32)]).reshape(EROWS, C)
    zeros = jnp.zeros((N, DH), jnp.float32)
    zeros16 = jnp.zeros((N, 16), jnp.float32)
    ones = jnp.ones((C, 16), jnp.float32)

    h0_lo, h0_hi = _enc_tc(features, W_enc)
    s0_lo, s0_hi, deg = _agg_deg(h0_lo, h0_hi, src2d, dst2d, zeros, zeros16,
                                 ones)
    x0, h1_lo, h1_hi = _mid_tc(s0_lo, s0_hi, deg, W_layers[0])
    s1_lo, s1_hi = _agg(h1_lo, h1_hi, src2d, dst2d, zeros)
    x1, h2_lo, h2_hi = _mid_tc(s1_lo, s1_hi, deg, W_layers[1])
    s2_lo, s2_hi = _agg(h2_lo, h2_hi, src2d, dst2d, zeros)
    logits = _fin_tc(s2_lo, s2_hi, deg, x0, x1, W_fc, b_fc.reshape(1, 40))
    return logits


# speedup vs baseline: 9.8945x; 1.0168x over previous
import functools

import jax
import jax.numpy as jnp
from jax import lax
from jax.experimental import pallas as pl
from jax.experimental.pallas import tpu as pltpu
from jax.experimental.pallas import tpu_sc as plsc

N = 10000
E = 320000
D = 128
DH = D // 2
T = 4
VTH = 1.0
STEP = 0.1

NC = 2
NS = 16
C = 128
NCH = -(-E // (NS * C))
E_PAD = NS * NCH * C
EROWS = E_PAD // C
NA = N + 16
ROWS_PT = (N // NS) // 8 * 8
ROWS_REM = N - NS * ROWS_PT
REM0 = NS * ROWS_PT

_MESH = plsc.VectorSubcoreMesh(
    core_axis_name="c", subcore_axis_name="s", num_cores=NC, num_subcores=NS
)


def _make_agg(with_deg: bool):
    out_type = [
        jax.ShapeDtypeStruct((N, DH), jnp.float32),
        jax.ShapeDtypeStruct((N, DH), jnp.float32),
    ]
    scratch = [
        pltpu.VMEM((NCH, C), jnp.int32),
        pltpu.VMEM((NCH, C), jnp.int32),
        pltpu.VMEM((C, DH), jnp.float32),
        pltpu.VMEM((C, DH), jnp.float32),
        pltpu.VMEM((C, DH), jnp.float32),
        pltpu.VMEM((C, DH), jnp.float32),
        pltpu.VMEM_SHARED((NA, DH), jnp.float32),
        pltpu.SemaphoreType.DMA,
        pltpu.SemaphoreType.DMA,
        pltpu.SemaphoreType.DMA,
        pltpu.SemaphoreType.DMA,
        pltpu.SemaphoreType.DMA,
        pltpu.SemaphoreType.DMA,
        pltpu.SemaphoreType.DMA,
        pltpu.SemaphoreType.DMA,
    ]
    if with_deg:
        out_type.append(jax.ShapeDtypeStruct((N, 16), jnp.float32))
        scratch += [
            pltpu.VMEM((C, 16), jnp.float32),
            pltpu.VMEM_SHARED((NA, 16), jnp.float32),
        ]

    @functools.partial(
        pl.kernel, out_type=out_type, mesh=_MESH, scratch_types=scratch,
        compiler_params=pltpu.CompilerParams(use_tc_tiling_on_sc=False),
    )
    def agg(*refs):
        if with_deg:
            (xlo_hbm, xhi_hbm, src_hbm, dst_hbm, zeros_hbm, zeros16_hbm,
             ones_hbm, outlo_hbm, outhi_hbm, deg_hbm,
             srci, dsti, rows0, rows1, rows2, rows3, acc,
             gsem0, gsem1, gsem2, gsem3, ssem0, ssem1, ssem2, ssem3,
             ones_v, dacc) = refs
        else:
            (xlo_hbm, xhi_hbm, src_hbm, dst_hbm, zeros_hbm,
             outlo_hbm, outhi_hbm,
             srci, dsti, rows0, rows1, rows2, rows3, acc,
             gsem0, gsem1, gsem2, gsem3, ssem0, ssem1, ssem2, ssem3) = refs

        c = lax.axis_index("c")
        s = lax.axis_index("s")
        r0 = s * ROWS_PT
        row_base = s * NCH
        deg_here = with_deg

        pltpu.async_copy(src_hbm.at[pl.ds(row_base, NCH)], srci, gsem0)
        pltpu.async_copy(dst_hbm.at[pl.ds(row_base, NCH)], dsti, gsem1)
        pltpu.async_copy(zeros_hbm.at[pl.ds(r0, ROWS_PT)],
                         acc.at[pl.ds(r0, ROWS_PT)], ssem0)
        if deg_here:
            pltpu.sync_copy(ones_hbm, ones_v)

            @pl.when(c == 0)
            def _zero_deg():
                pltpu.async_copy(zeros16_hbm.at[pl.ds(r0, ROWS_PT)],
                                 dacc.at[pl.ds(r0, ROWS_PT)], ssem1)

        @pl.when(s == NS - 1)
        def _zero_rem():
            pltpu.async_copy(zeros_hbm.at[pl.ds(REM0, ROWS_REM)],
                             acc.at[pl.ds(REM0, ROWS_REM)], ssem0)
            if with_deg:
                @pl.when(c == 0)
                def _zero_deg_rem():
                    pltpu.async_copy(zeros16_hbm.at[pl.ds(REM0, ROWS_REM)],
                                     dacc.at[pl.ds(REM0, ROWS_REM)], ssem1)

        pltpu.make_async_copy(src_hbm.at[pl.ds(row_base, NCH)], srci,
                              gsem0).wait()
        pltpu.make_async_copy(dst_hbm.at[pl.ds(row_base, NCH)], dsti,
                              gsem1).wait()

        def run_core(x_hbm):
            rows = [rows0, rows1, rows2, rows3]
            gsem = [gsem0, gsem1, gsem2, gsem3]
            ssem = [ssem0, ssem1, ssem2, ssem3]

            def gather(k, b):
                pltpu.async_copy(x_hbm.at[srci.at[k]], rows[b], gsem[b])

            def gwait(k, b):
                pltpu.make_async_copy(x_hbm.at[srci.at[k]], rows[b],
                                      gsem[b]).wait()

            def scat(k, b):
                pltpu.async_copy(rows[b], acc.at[dsti.at[k]], ssem[b],
                                 add=True)
                if deg_here:
                    @pl.when(c == 0)
                    def _deg_add():
                        pltpu.sync_copy(ones_v, dacc.at[dsti.at[k]], add=True)

            def swait(k, b):
                pltpu.make_async_copy(rows[b], acc.at[dsti.at[k]],
                                      ssem[b]).wait()

            gather(0, 0)
            gather(1, 1)
            gather(2, 2)
            gather(3, 3)
            pltpu.make_async_copy(zeros_hbm.at[pl.ds(r0, ROWS_PT)],
                                  acc.at[pl.ds(r0, ROWS_PT)], ssem0).wait()

            @pl.when(s == NS - 1)
            def _zero_rem_wait():
                pltpu.make_async_copy(zeros_hbm.at[pl.ds(REM0, ROWS_REM)],
                                      acc.at[pl.ds(REM0, ROWS_REM)],
                                      ssem0).wait()

            if deg_here:
                @pl.when(c == 0)
                def _zero_deg_wait():
                    pltpu.make_async_copy(zeros16_hbm.at[pl.ds(r0, ROWS_PT)],
                                          dacc.at[pl.ds(r0, ROWS_PT)],
                                          ssem1).wait()

                    @pl.when(s == NS - 1)
                    def _zero_deg_rem_wait():
                        pltpu.make_async_copy(
                            zeros16_hbm.at[pl.ds(REM0, ROWS_REM)],
                            dacc.at[pl.ds(REM0, ROWS_REM)], ssem1).wait()

            plsc.subcore_barrier()
            gwait(0, 0)
            scat(0, 0)
            gwait(1, 1)
            scat(1, 1)

            def quad(j, _):
                for b in range(4):
                    i = j * 4 + b
                    swait(i - 4, b)
                    gather(i, b)
                    b2 = (b + 2) % 4
                    gwait(i - 2, b2)
                    scat(i - 2, b2)
                return 0

            lax.fori_loop(1, NCH // 4, quad, 0)
            swait(NCH - 5, 0)
            gather(NCH - 1, 0)
            gwait(NCH - 3, 2)
            scat(NCH - 3, 2)
            gwait(NCH - 2, 3)
            scat(NCH - 2, 3)
            gwait(NCH - 1, 0)
            scat(NCH - 1, 0)
            swait(NCH - 4, 1)
            swait(NCH - 3, 2)
            swait(NCH - 2, 3)
            swait(NCH - 1, 0)

        @pl.when(c == 0)
        def _core0():
            run_core(xlo_hbm)

        @pl.when(c == 1)
        def _core1():
            run_core(xhi_hbm)

        plsc.subcore_barrier()

        def write_out(out_hbm):
            pltpu.sync_copy(acc.at[pl.ds(r0, ROWS_PT)],
                            out_hbm.at[pl.ds(r0, ROWS_PT)])

            @pl.when(s == NS - 1)
            def _wb_rem():
                pltpu.sync_copy(acc.at[pl.ds(REM0, ROWS_REM)],
                                out_hbm.at[pl.ds(REM0, ROWS_REM)])

        @pl.when(c == 0)
        def _wb0():
            write_out(outlo_hbm)
            if deg_here:
                pltpu.sync_copy(dacc.at[pl.ds(r0, ROWS_PT)],
                                deg_hbm.at[pl.ds(r0, ROWS_PT)])

                @pl.when(s == NS - 1)
                def _wb_deg_rem():
                    pltpu.sync_copy(dacc.at[pl.ds(REM0, ROWS_REM)],
                                    deg_hbm.at[pl.ds(REM0, ROWS_REM)])

        @pl.when(c == 1)
        def _wb1():
            write_out(outhi_hbm)

    return agg


_agg_deg = _make_agg(True)
_agg = _make_agg(False)


B = 1000


def _if_rate(lo_ref, hi_ref, deg_ref):
    t = jnp.concatenate((lo_ref[...], hi_ref[...]), axis=1)
    t = t / jnp.maximum(deg_ref[:, 0:1], 1.0)
    v = jnp.zeros_like(t)
    ssum = jnp.zeros_like(t)
    for _ in range(T):
        v = v + t
        sp = (v >= VTH).astype(t.dtype)
        v = v - sp * VTH
        ssum = ssum + sp
    return ssum * (1.0 / T)


def _enc_body(f_ref, w_ref, hlo_ref, hhi_ref):
    h = jnp.dot(f_ref[...], w_ref[...], preferred_element_type=jnp.float32)
    hlo_ref[...] = h[:, :DH]
    hhi_ref[...] = h[:, DH:]


def _mid_body(lo_ref, hi_ref, deg_ref, w_ref, x_ref, hlo_ref, hhi_ref):
    x = _if_rate(lo_ref, hi_ref, deg_ref)
    x_ref[...] = x
    h = jnp.dot(x, w_ref[...], preferred_element_type=jnp.float32)
    hlo_ref[...] = h[:, :DH]
    hhi_ref[...] = h[:, DH:]


def _fin_body(lo_ref, hi_ref, deg_ref, x0_ref, x1_ref, wfc_ref, bfc_ref,
              out_ref):
    x2 = _if_rate(lo_ref, hi_ref, deg_ref)
    z = (x0_ref[...] + x1_ref[...] + x2) * STEP
    out_ref[...] = jnp.dot(z, wfc_ref[...],
                           preferred_element_type=jnp.float32) + bfc_ref[...]


_half_spec = pl.BlockSpec((B, DH), lambda i: (i, 0))
_full_spec = pl.BlockSpec((B, D), lambda i: (i, 0))
_deg_spec = pl.BlockSpec((B, 16), lambda i: (i, 0))
_w_spec = pl.BlockSpec((D, D), lambda i: (0, 0))
_half_out = [
    jax.ShapeDtypeStruct((N, DH), jnp.float32),
    jax.ShapeDtypeStruct((N, DH), jnp.float32),
]

_enc_tc = pl.pallas_call(
    _enc_body,
    grid=(N // B,),
    in_specs=[_full_spec, _w_spec],
    out_specs=[_half_spec, _half_spec],
    out_shape=_half_out,
)

_mid_tc = pl.pallas_call(
    _mid_body,
    grid=(N // B,),
    in_specs=[_half_spec, _half_spec, _deg_spec, _w_spec],
    out_specs=[_full_spec, _half_spec, _half_spec],
    out_shape=[jax.ShapeDtypeStruct((N, D), jnp.float32)] + _half_out,
)

_fin_tc = pl.pallas_call(
    _fin_body,
    grid=(N // B,),
    in_specs=[
        _half_spec,
        _half_spec,
        _deg_spec,
        _full_spec,
        _full_spec,
        pl.BlockSpec((D, 40), lambda i: (0, 0)),
        pl.BlockSpec((1, 40), lambda i: (0, 0)),
    ],
    out_specs=pl.BlockSpec((B, 40), lambda i: (i, 0)),
    out_shape=jax.ShapeDtypeStruct((N, 40), jnp.float32),
)


def kernel(features, edge_index, W_enc, W_layers, W_fc, b_fc):
    src2d = jnp.concatenate(
        [edge_index[0], jnp.zeros((E_PAD - E,), jnp.int32)]).reshape(EROWS, C)
    dst2d = jnp.concatenate(
        [edge_index[1], jnp.full((E_PAD - E,), N, jnp.int32)]).reshape(EROWS, C)
    zeros = jnp.zeros((N, DH), jnp.float32)
    zeros16 = jnp.zeros((N, 16), jnp.float32)
    ones = jnp.ones((C, 16), jnp.float32)

    h0_lo, h0_hi = _enc_tc(features, W_enc)
    s0_lo, s0_hi, deg = _agg_deg(h0_lo, h0_hi, src2d, dst2d, zeros, zeros16,
                                 ones)
    x0, h1_lo, h1_hi = _mid_tc(s0_lo, s0_hi, deg, W_layers[0])
    s1_lo, s1_hi = _agg(h1_lo, h1_hi, src2d, dst2d, zeros)
    x1, h2_lo, h2_hi = _mid_tc(s1_lo, s1_hi, deg, W_layers[1])
    s2_lo, s2_hi = _agg(h2_lo, h2_hi, src2d, dst2d, zeros)
    logits = _fin_tc(s2_lo, s2_hi, deg, x0, x1, W_fc, b_fc.reshape(1, 40))
    return logits


# speedup vs baseline: 10.0078x; 1.0114x over previous
import functools

import jax
import jax.numpy as jnp
from jax import lax
from jax.experimental import pallas as pl
from jax.experimental.pallas import tpu as pltpu
from jax.experimental.pallas import tpu_sc as plsc

N = 10000
E = 320000
D = 128
DH = D // 2
T = 4
VTH = 1.0
STEP = 0.1

NC = 2
NS = 16
C = 128
NCH = -(-E // (NS * C))
E_PAD = NS * NCH * C
EROWS = E_PAD // C
NA = N + 16
ROWS_PT = (N // NS) // 8 * 8
ROWS_REM = N - NS * ROWS_PT
REM0 = NS * ROWS_PT

_MESH = plsc.VectorSubcoreMesh(
    core_axis_name="c", subcore_axis_name="s", num_cores=NC, num_subcores=NS
)


def _make_agg(with_deg: bool):
    out_type = [
        jax.ShapeDtypeStruct((N, DH), jnp.float32),
        jax.ShapeDtypeStruct((N, DH), jnp.float32),
    ]
    scratch = [
        pltpu.VMEM((NCH, C), jnp.int32),
        pltpu.VMEM((NCH, C), jnp.int32),
        pltpu.VMEM((C, DH), jnp.float32),
        pltpu.VMEM((C, DH), jnp.float32),
        pltpu.VMEM((C, DH), jnp.float32),
        pltpu.VMEM((C, DH), jnp.float32),
        pltpu.VMEM_SHARED((NA, DH), jnp.float32),
        pltpu.SemaphoreType.DMA,
        pltpu.SemaphoreType.DMA,
        pltpu.SemaphoreType.DMA,
        pltpu.SemaphoreType.DMA,
        pltpu.SemaphoreType.DMA,
        pltpu.SemaphoreType.DMA,
        pltpu.SemaphoreType.DMA,
        pltpu.SemaphoreType.DMA,
    ]
    if with_deg:
        out_type.append(jax.ShapeDtypeStruct((N, 16), jnp.float32))
        scratch += [
            pltpu.VMEM((C, 16), jnp.float32),
            pltpu.VMEM_SHARED((NA, 16), jnp.float32),
        ]

    @functools.partial(
        pl.kernel, out_type=out_type, mesh=_MESH, scratch_types=scratch,
        compiler_params=pltpu.CompilerParams(use_tc_tiling_on_sc=False),
    )
    def agg(*refs):
        if with_deg:
            (xlo_hbm, xhi_hbm, edges_hbm, zeros_hbm, zeros16_hbm,
             ones_hbm, outlo_hbm, outhi_hbm, deg_hbm,
             srci, dsti, rows0, rows1, rows2, rows3, acc,
             gsem0, gsem1, gsem2, gsem3, ssem0, ssem1, ssem2, ssem3,
             ones_v, dacc) = refs
        else:
            (xlo_hbm, xhi_hbm, edges_hbm, zeros_hbm,
             outlo_hbm, outhi_hbm,
             srci, dsti, rows0, rows1, rows2, rows3, acc,
             gsem0, gsem1, gsem2, gsem3, ssem0, ssem1, ssem2, ssem3) = refs

        c = lax.axis_index("c")
        s = lax.axis_index("s")
        r0 = s * ROWS_PT
        row_base = s * NCH
        deg_here = with_deg

        pltpu.async_copy(edges_hbm.at[0].at[pl.ds(row_base, NCH)], srci, gsem0)
        pltpu.async_copy(edges_hbm.at[1].at[pl.ds(row_base, NCH)], dsti, gsem1)
        pltpu.async_copy(zeros_hbm.at[pl.ds(r0, ROWS_PT)],
                         acc.at[pl.ds(r0, ROWS_PT)], ssem0)
        if deg_here:
            pltpu.sync_copy(ones_hbm, ones_v)

            @pl.when(c == 0)
            def _zero_deg():
                pltpu.async_copy(zeros16_hbm.at[pl.ds(r0, ROWS_PT)],
                                 dacc.at[pl.ds(r0, ROWS_PT)], ssem1)

        @pl.when(s == NS - 1)
        def _zero_rem():
            pltpu.async_copy(zeros_hbm.at[pl.ds(REM0, ROWS_REM)],
                             acc.at[pl.ds(REM0, ROWS_REM)], ssem0)
            if with_deg:
                @pl.when(c == 0)
                def _zero_deg_rem():
                    pltpu.async_copy(zeros16_hbm.at[pl.ds(REM0, ROWS_REM)],
                                     dacc.at[pl.ds(REM0, ROWS_REM)], ssem1)

        pltpu.make_async_copy(edges_hbm.at[0].at[pl.ds(row_base, NCH)], srci,
                              gsem0).wait()
        pltpu.make_async_copy(edges_hbm.at[1].at[pl.ds(row_base, NCH)], dsti,
                              gsem1).wait()

        def run_core(x_hbm):
            rows = [rows0, rows1, rows2, rows3]
            gsem = [gsem0, gsem1, gsem2, gsem3]
            ssem = [ssem0, ssem1, ssem2, ssem3]

            def gather(k, b):
                pltpu.async_copy(x_hbm.at[srci.at[k]], rows[b], gsem[b])

            def gwait(k, b):
                pltpu.make_async_copy(x_hbm.at[srci.at[k]], rows[b],
                                      gsem[b]).wait()

            def scat(k, b):
                pltpu.async_copy(rows[b], acc.at[dsti.at[k]], ssem[b],
                                 add=True)
                if deg_here:
                    @pl.when(c == 0)
                    def _deg_add():
                        pltpu.sync_copy(ones_v, dacc.at[dsti.at[k]], add=True)

            def swait(k, b):
                pltpu.make_async_copy(rows[b], acc.at[dsti.at[k]],
                                      ssem[b]).wait()

            gather(0, 0)
            gather(1, 1)
            gather(2, 2)
            gather(3, 3)
            pltpu.make_async_copy(zeros_hbm.at[pl.ds(r0, ROWS_PT)],
                                  acc.at[pl.ds(r0, ROWS_PT)], ssem0).wait()

            @pl.when(s == NS - 1)
            def _zero_rem_wait():
                pltpu.make_async_copy(zeros_hbm.at[pl.ds(REM0, ROWS_REM)],
                                      acc.at[pl.ds(REM0, ROWS_REM)],
                                      ssem0).wait()

            if deg_here:
                @pl.when(c == 0)
                def _zero_deg_wait():
                    pltpu.make_async_copy(zeros16_hbm.at[pl.ds(r0, ROWS_PT)],
                                          dacc.at[pl.ds(r0, ROWS_PT)],
                                          ssem1).wait()

                    @pl.when(s == NS - 1)
                    def _zero_deg_rem_wait():
                        pltpu.make_async_copy(
                            zeros16_hbm.at[pl.ds(REM0, ROWS_REM)],
                            dacc.at[pl.ds(REM0, ROWS_REM)], ssem1).wait()

            plsc.subcore_barrier()
            gwait(0, 0)
            scat(0, 0)
            gwait(1, 1)
            scat(1, 1)

            def quad(j, _):
                for b in range(4):
                    i = j * 4 + b
                    swait(i - 4, b)
                    gather(i, b)
                    b2 = (b + 2) % 4
                    gwait(i - 2, b2)
                    scat(i - 2, b2)
                return 0

            lax.fori_loop(1, NCH // 4, quad, 0)
            swait(NCH - 5, 0)
            gather(NCH - 1, 0)
            gwait(NCH - 3, 2)
            scat(NCH - 3, 2)
            gwait(NCH - 2, 3)
            scat(NCH - 2, 3)
            gwait(NCH - 1, 0)
            scat(NCH - 1, 0)
            swait(NCH - 4, 1)
            swait(NCH - 3, 2)
            swait(NCH - 2, 3)
            swait(NCH - 1, 0)

        @pl.when(c == 0)
        def _core0():
            run_core(xlo_hbm)

        @pl.when(c == 1)
        def _core1():
            run_core(xhi_hbm)

        plsc.subcore_barrier()

        def write_out(out_hbm):
            pltpu.sync_copy(acc.at[pl.ds(r0, ROWS_PT)],
                            out_hbm.at[pl.ds(r0, ROWS_PT)])

            @pl.when(s == NS - 1)
            def _wb_rem():
                pltpu.sync_copy(acc.at[pl.ds(REM0, ROWS_REM)],
                                out_hbm.at[pl.ds(REM0, ROWS_REM)])

        @pl.when(c == 0)
        def _wb0():
            write_out(outlo_hbm)
            if deg_here:
                pltpu.sync_copy(dacc.at[pl.ds(r0, ROWS_PT)],
                                deg_hbm.at[pl.ds(r0, ROWS_PT)])

                @pl.when(s == NS - 1)
                def _wb_deg_rem():
                    pltpu.sync_copy(dacc.at[pl.ds(REM0, ROWS_REM)],
                                    deg_hbm.at[pl.ds(REM0, ROWS_REM)])

        @pl.when(c == 1)
        def _wb1():
            write_out(outhi_hbm)

    return agg


_agg_deg = _make_agg(True)
_agg = _make_agg(False)


B = 2000


def _if_rate(lo_ref, hi_ref, deg_ref):
    t = jnp.concatenate((lo_ref[...], hi_ref[...]), axis=1)
    t = t / jnp.maximum(deg_ref[:, 0:1], 1.0)
    v = jnp.zeros_like(t)
    ssum = jnp.zeros_like(t)
    for _ in range(T):
        v = v + t
        sp = (v >= VTH).astype(t.dtype)
        v = v - sp * VTH
        ssum = ssum + sp
    return ssum * (1.0 / T)


def _enc_body(f_ref, w_ref, hlo_ref, hhi_ref):
    h = jnp.dot(f_ref[...], w_ref[...], preferred_element_type=jnp.float32)
    hlo_ref[...] = h[:, :DH]
    hhi_ref[...] = h[:, DH:]


def _mid_body(lo_ref, hi_ref, deg_ref, w_ref, x_ref, hlo_ref, hhi_ref):
    x = _if_rate(lo_ref, hi_ref, deg_ref)
    x_ref[...] = x
    h = jnp.dot(x, w_ref[...], preferred_element_type=jnp.float32)
    hlo_ref[...] = h[:, :DH]
    hhi_ref[...] = h[:, DH:]


def _fin_body(lo_ref, hi_ref, deg_ref, x0_ref, x1_ref, wfc_ref, bfc_ref,
              out_ref):
    x2 = _if_rate(lo_ref, hi_ref, deg_ref)
    z = (x0_ref[...] + x1_ref[...] + x2) * STEP
    out_ref[...] = jnp.dot(z, wfc_ref[...],
                           preferred_element_type=jnp.float32) + bfc_ref[...]


_half_spec = pl.BlockSpec((B, DH), lambda i: (i, 0))
_full_spec = pl.BlockSpec((B, D), lambda i: (i, 0))
_deg_spec = pl.BlockSpec((B, 16), lambda i: (i, 0))
_w_spec = pl.BlockSpec((D, D), lambda i: (0, 0))
_half_out = [
    jax.ShapeDtypeStruct((N, DH), jnp.float32),
    jax.ShapeDtypeStruct((N, DH), jnp.float32),
]

_enc_tc = pl.pallas_call(
    _enc_body,
    grid=(N // B,),
    in_specs=[_full_spec, _w_spec],
    out_specs=[_half_spec, _half_spec],
    out_shape=_half_out,
)

_mid_tc = pl.pallas_call(
    _mid_body,
    grid=(N // B,),
    in_specs=[_half_spec, _half_spec, _deg_spec, _w_spec],
    out_specs=[_full_spec, _half_spec, _half_spec],
    out_shape=[jax.ShapeDtypeStruct((N, D), jnp.float32)] + _half_out,
)

_fin_tc = pl.pallas_call(
    _fin_body,
    grid=(N // B,),
    in_specs=[
        _half_spec,
        _half_spec,
        _deg_spec,
        _full_spec,
        _full_spec,
        pl.BlockSpec((D, 40), lambda i: (0, 0)),
        pl.BlockSpec((1, 40), lambda i: (0, 0)),
    ],
    out_specs=pl.BlockSpec((B, 40), lambda i: (i, 0)),
    out_shape=jax.ShapeDtypeStruct((N, 40), jnp.float32),
)


def kernel(features, edge_index, W_enc, W_layers, W_fc, b_fc):
    ep = jnp.pad(edge_index, ((0, 0), (0, E_PAD - E)))
    ep = ep.at[1, E:].set(N)
    edges3d = ep.reshape(2, EROWS, C)
    zeros = jnp.zeros((N, DH), jnp.float32)
    zeros16 = jnp.zeros((N, 16), jnp.float32)
    ones = jnp.ones((C, 16), jnp.float32)

    h0_lo, h0_hi = _enc_tc(features, W_enc)
    s0_lo, s0_hi, deg = _agg_deg(h0_lo, h0_hi, edges3d, zeros, zeros16,
                                 ones)
    x0, h1_lo, h1_hi = _mid_tc(s0_lo, s0_hi, deg, W_layers[0])
    s1_lo, s1_hi = _agg(h1_lo, h1_hi, edges3d, zeros)
    x1, h2_lo, h2_hi = _mid_tc(s1_lo, s1_hi, deg, W_layers[1])
    s2_lo, s2_hi = _agg(h2_lo, h2_hi, edges3d, zeros)
    logits = _fin_tc(s2_lo, s2_hi, deg, x0, x1, W_fc, b_fc.reshape(1, 40))
    return logits
